# Initial kernel scaffold; baseline (speedup 1.0000x reference)
import jax
import jax.numpy as jnp
from jax.experimental import pallas as pl


def kernel(seq, W, U):
    raise NotImplementedError("write your pallas kernel here")


# speedup vs baseline: 1.5094x; 1.5094x over previous
import functools

import jax
import jax.numpy as jnp
from jax import lax
from jax.experimental import pallas as pl
from jax.experimental.pallas import tpu as pltpu
from jax.experimental.pallas import tpu_sc as plsc

NC = 2
NS = 16
NW = NC * NS
LANES = 16

EMB = 64
E_SL = EMB // LANES
CHUNK = 64


def _region_encode(seq_flat, W, U, *, B, L, R):
    TOK = B * L
    per_w = TOK // NW
    n_chunks = per_w // CHUNK
    RAD = (R - 1) // 2

    mesh = plsc.VectorSubcoreMesh(
        core_axis_name="c", subcore_axis_name="s", num_cores=NC, num_subcores=NS
    )

    @functools.partial(
        pl.kernel,
        out_type=jax.ShapeDtypeStruct((TOK, EMB), jnp.float32),
        mesh=mesh,
        compiler_params=pltpu.CompilerParams(
            needs_layout_passes=False, use_tc_tiling_on_sc=False
        ),
        scratch_types=[
            pltpu.VMEM((per_w,), jnp.int32),
            pltpu.VMEM((CHUNK,), jnp.int32),
            pltpu.VMEM((R, CHUNK), jnp.int32),
            pltpu.VMEM((CHUNK, EMB), jnp.float32),
            pltpu.VMEM((R, CHUNK, EMB), jnp.float32),
            pltpu.VMEM((CHUNK, EMB), jnp.float32),
            pltpu.SemaphoreType.DMA,
            pltpu.SemaphoreType.DMA,
        ],
    )
    def k(seq_hbm, W_hbm, U_hbm, out_hbm,
          seq_v, w_idx, u_idx, w_rows, u_rows, out_v, semw, semu):
        wid = lax.axis_index("s") * NC + lax.axis_index("c")
        base = wid * per_w
        pltpu.sync_copy(seq_hbm.at[pl.ds(base, per_w)], seq_v)

        lane = lax.broadcasted_iota(jnp.int32, (LANES,), 0)

        @pl.loop(0, n_chunks)
        def chunk_loop(kk):
            c0 = kk * CHUNK
            for j in range(CHUNK // LANES):
                p0 = c0 + j * LANES
                p = p0 + lane
                l = lax.rem(p, L)
                tok = seq_v[pl.ds(p0, LANES)]
                w_idx[pl.ds(j * LANES, LANES)] = tok
                for i in range(R):
                    d = i - RAD
                    if d == 0:
                        ntok = tok
                    else:
                        lv = l + d
                        nb = jnp.clip(p + d, 0, per_w - 1)
                        g = plsc.load_gather(seq_v, [nb])
                        valid = (lv >= 0) & (lv <= L - 1)
                        ntok = jnp.where(valid, g, 0)
                    u_idx[i, pl.ds(j * LANES, LANES)] = ntok * R + i

            cw = pltpu.async_copy(W_hbm.at[w_idx], w_rows, semw)
            cus = [
                pltpu.async_copy(U_hbm.at[u_idx.at[i]], u_rows.at[i], semu)
                for i in range(R)
            ]
            cw.wait()
            for cu in cus:
                cu.wait()

            @pl.loop(0, CHUNK // LANES)
            def grp_loop(j):
                tok_vec = w_idx[pl.ds(j * LANES, LANES)]
                mvec = jnp.where(tok_vec != 0, 1.0, 0.0).astype(jnp.float32)
                for cl in range(LANES):
                    c = j * LANES + cl
                    maskf = mvec[cl]
                    for e in range(E_SL):
                        es = pl.ds(e * LANES, LANES)
                        w_e = w_rows[c, es]
                        acc = u_rows[0, c, es] * w_e
                        for i in range(1, R):
                            acc = jnp.maximum(acc, u_rows[i, c, es] * w_e)
                        out_v[c, es] = acc * maskf

            pltpu.sync_copy(out_v, out_hbm.at[pl.ds(base + c0, CHUNK)])

    return k(seq_flat, W, U)


def kernel(seq, W, U):
    B, L, _ = seq.shape
    R = U.shape[0] // W.shape[0]
    out = _region_encode(seq.reshape(B * L), W, U, B=B, L=L, R=R)
    return out.reshape(B, L, 1, EMB)
